# Initial kernel scaffold; baseline (speedup 1.0000x reference)
import functools

import jax
import jax.numpy as jnp
from jax import lax
from jax.experimental import pallas as pl
from jax.experimental.pallas import tpu as pltpu
from jax.experimental.pallas import tpu_sc as plsc

_NC = 2
_NS = 16
_NW = _NC * _NS
_EU = 128


def _make_seg_sum(n, e, dp, name):
    assert e % _EU == 0 and n % _NS == 0
    units = e // _EU
    base_units = units // _NW
    rem = units - base_units * _NW
    rpt = n // _NS
    wch = 125
    assert rpt % wch == 0
    mesh = plsc.VectorSubcoreMesh(core_axis_name="c", subcore_axis_name="s")

    @functools.partial(
        pl.kernel,
        out_type=jax.ShapeDtypeStruct((_NC, n, dp), jnp.float32),
        mesh=mesh,
        scratch_types=[
            pltpu.VMEM((_EU,), jnp.int32),
            pltpu.VMEM((_EU,), jnp.int32),
            pltpu.VMEM((_EU, dp), jnp.float32),
            pltpu.VMEM_SHARED((n, dp), jnp.float32),
            pltpu.SemaphoreType.DMA,
        ],
        name=name,
    )
    def seg_sum(feat_hbm, src_hbm, dst_hbm, zero_hbm, out_hbm,
                sidx, didx, rows, acc, sem):
        c = lax.axis_index("c")
        s = lax.axis_index("s")
        w = s * _NC + c
        pltpu.sync_copy(zero_hbm, acc.at[pl.ds(s * rpt, rpt)])
        plsc.subcore_barrier()

        nb = base_units + jnp.where(w < rem, 1, 0)
        u0 = w * base_units + jnp.minimum(w, rem)

        def body(i, carry):
            eoff = (u0 + i) * _EU
            pltpu.sync_copy(src_hbm.at[pl.ds(eoff, _EU)], sidx)
            pltpu.sync_copy(dst_hbm.at[pl.ds(eoff, _EU)], didx)
            pltpu.async_copy(feat_hbm.at[sidx], rows, sem).wait()
            pltpu.sync_copy(rows, acc.at[didx], add=True)
            return carry

        lax.fori_loop(0, nb, body, 0)
        plsc.subcore_barrier()
        for k in range(rpt // wch):
            r0 = s * rpt + k * wch
            pltpu.sync_copy(acc.at[pl.ds(r0, wch)], out_hbm.at[c, pl.ds(r0, wch)])

    return seg_sum


def _tc_dense(x, sum1, w1l, w1r, w2lp, w2rp, b1, b2p, n, blk):
    d = x.shape[1]
    grid = (n // blk,)

    def body(x_ref, s1_ref, w1l_ref, w1r_ref, w2l_ref, w2r_ref, b1_ref, b2_ref,
             y2_ref, hr_ref, rc_ref):
        sums = s1_ref[0] + s1_ref[1]
        agg = sums[:, :d]
        cnt = sums[:, d:d + 1]
        recip = 1.0 / jnp.maximum(cnt, 1.0)
        aggm = agg * recip
        h = aggm @ w1l_ref[...].T + x_ref[...] @ w1r_ref[...].T + b1_ref[...]
        h = jnp.maximum(h, 0.0)
        y2_ref[...] = h @ w2l_ref[...].T
        hr_ref[...] = h @ w2r_ref[...].T + b2_ref[...]
        rc_ref[...] = jnp.broadcast_to(recip, (blk, 16))

    dp = sum1.shape[2]
    return pl.pallas_call(
        body,
        grid=grid,
        in_specs=[
            pl.BlockSpec((blk, d), lambda i: (i, 0)),
            pl.BlockSpec((2, blk, dp), lambda i: (0, i, 0)),
            pl.BlockSpec(w1l.shape, lambda i: (0, 0)),
            pl.BlockSpec(w1r.shape, lambda i: (0, 0)),
            pl.BlockSpec(w2lp.shape, lambda i: (0, 0)),
            pl.BlockSpec(w2rp.shape, lambda i: (0, 0)),
            pl.BlockSpec(b1.shape, lambda i: (0, 0)),
            pl.BlockSpec(b2p.shape, lambda i: (0, 0)),
        ],
        out_specs=[
            pl.BlockSpec((blk, 16), lambda i: (i, 0)),
            pl.BlockSpec((blk, 16), lambda i: (i, 0)),
            pl.BlockSpec((blk, 16), lambda i: (i, 0)),
        ],
        out_shape=[
            jax.ShapeDtypeStruct((n, 16), jnp.float32),
            jax.ShapeDtypeStruct((n, 16), jnp.float32),
            jax.ShapeDtypeStruct((n, 16), jnp.float32),
        ],
    )(x, sum1, w1l, w1r, w2lp, w2rp, b1, b2p)


def _tc_combine(sum2, rc16, hr, n):
    def body(s2_ref, rc_ref, hr_ref, o_ref):
        o_ref[...] = (s2_ref[0] + s2_ref[1]) * rc_ref[...] + hr_ref[...]

    return pl.pallas_call(
        body,
        grid=(1,),
        in_specs=[
            pl.BlockSpec((2, n, 16), lambda i: (0, 0, 0)),
            pl.BlockSpec((n, 16), lambda i: (0, 0)),
            pl.BlockSpec((n, 16), lambda i: (0, 0)),
        ],
        out_specs=pl.BlockSpec((n, 16), lambda i: (0, 0)),
        out_shape=jax.ShapeDtypeStruct((n, 16), jnp.float32),
    )(sum2, rc16, hr)


def kernel(x, edge_index, W1l, b1, W1r, W2l, b2, W2r):
    n, d = x.shape
    e = edge_index.shape[1]
    o = W2l.shape[0]
    dp = d + 16

    src = edge_index[0]
    dst = edge_index[1]

    xx = jnp.concatenate(
        [x, jnp.ones((n, 1), jnp.float32), jnp.zeros((n, 15), jnp.float32)], axis=1)
    zero1 = jnp.zeros((n // _NS, dp), jnp.float32)
    zero2 = jnp.zeros((n // _NS, 16), jnp.float32)
    w2lp = jnp.zeros((16, d), jnp.float32).at[:o].set(W2l)
    w2rp = jnp.zeros((16, d), jnp.float32).at[:o].set(W2r)
    b2p = jnp.zeros((1, 16), jnp.float32).at[0, :o].set(b2)
    b1r = b1.reshape(1, d)

    seg1 = _make_seg_sum(n, e, dp, "sage_seg_sum_l1")
    seg2 = _make_seg_sum(n, e, 16, "sage_seg_sum_l2")

    sum1 = seg1(xx, src, dst, zero1)
    y2, hr, rc16 = _tc_dense(x, sum1, W1l, W1r, w2lp, w2rp, b1r, b2p, n, 1000)
    sum2 = seg2(y2, src, dst, zero2)
    outp = _tc_combine(sum2, rc16, hr, n)
    return outp[:, :o]


# speedup vs baseline: 6.6963x; 6.6963x over previous
import functools

import jax
import jax.numpy as jnp
from jax import lax
from jax.experimental import pallas as pl
from jax.experimental.pallas import tpu as pltpu
from jax.experimental.pallas import tpu_sc as plsc

_NC = 2
_NS = 16
_NW = _NC * _NS
_EU = 128


def _make_seg_sum(n, e, dp, name):
    assert e % _EU == 0
    units = e // _EU
    base_units = units // _NW
    rem = units - base_units * _NW
    npad = -(-n // _EU) * _EU
    rpt = npad // _NS
    mesh = plsc.VectorSubcoreMesh(core_axis_name="c", subcore_axis_name="s")

    @functools.partial(
        pl.kernel,
        out_type=jax.ShapeDtypeStruct((_NC, npad, dp), jnp.float32),
        mesh=mesh,
        scratch_types=[
            pltpu.VMEM((_EU,), jnp.int32),
            pltpu.VMEM((_EU,), jnp.int32),
            pltpu.VMEM((_EU, dp), jnp.float32),
            pltpu.VMEM_SHARED((npad, dp), jnp.float32),
            pltpu.SemaphoreType.DMA,
        ],
        compiler_params=pltpu.CompilerParams(use_tc_tiling_on_sc=False),
        name=name,
    )
    def seg_sum(feat_hbm, src_hbm, dst_hbm, zero_hbm, out_hbm,
                sidx, didx, rows, acc, sem):
        c = lax.axis_index("c")
        s = lax.axis_index("s")
        w = s * _NC + c
        pltpu.sync_copy(zero_hbm, acc.at[pl.ds(s * rpt, rpt)])
        plsc.subcore_barrier()

        nb = base_units + jnp.where(w < rem, 1, 0)
        u0 = w * base_units + jnp.minimum(w, rem)

        def body(i, carry):
            eoff = (u0 + i) * _EU
            pltpu.sync_copy(src_hbm.at[pl.ds(eoff, _EU)], sidx)
            pltpu.sync_copy(dst_hbm.at[pl.ds(eoff, _EU)], didx)
            pltpu.async_copy(feat_hbm.at[sidx], rows, sem).wait()
            pltpu.sync_copy(rows, acc.at[didx], add=True)
            return carry

        lax.fori_loop(0, nb, body, 0)
        plsc.subcore_barrier()
        pltpu.sync_copy(acc.at[pl.ds(s * rpt, rpt)],
                        out_hbm.at[c, pl.ds(s * rpt, rpt)])

    return seg_sum


def _tc_dense(x, sum1, w1l, w1r, w2lp, w2rp, b1, b2p, n, blk):
    d = x.shape[1]
    grid = (n // blk,)

    def body(x_ref, s1_ref, w1l_ref, w1r_ref, w2l_ref, w2r_ref, b1_ref, b2_ref,
             y2_ref, hr_ref, rc_ref):
        sums = s1_ref[0] + s1_ref[1]
        agg = sums[:, :d]
        cnt = sums[:, d:d + 1]
        recip = 1.0 / jnp.maximum(cnt, 1.0)
        aggm = agg * recip
        h = aggm @ w1l_ref[...].T + x_ref[...] @ w1r_ref[...].T + b1_ref[...]
        h = jnp.maximum(h, 0.0)
        y2_ref[...] = h @ w2l_ref[...].T
        hr_ref[...] = h @ w2r_ref[...].T + b2_ref[...]
        rc_ref[...] = jnp.broadcast_to(recip, (blk, 16))

    dp = sum1.shape[2]
    return pl.pallas_call(
        body,
        grid=grid,
        in_specs=[
            pl.BlockSpec((blk, d), lambda i: (i, 0)),
            pl.BlockSpec((2, blk, dp), lambda i: (0, i, 0)),
            pl.BlockSpec(w1l.shape, lambda i: (0, 0)),
            pl.BlockSpec(w1r.shape, lambda i: (0, 0)),
            pl.BlockSpec(w2lp.shape, lambda i: (0, 0)),
            pl.BlockSpec(w2rp.shape, lambda i: (0, 0)),
            pl.BlockSpec(b1.shape, lambda i: (0, 0)),
            pl.BlockSpec(b2p.shape, lambda i: (0, 0)),
        ],
        out_specs=[
            pl.BlockSpec((blk, 16), lambda i: (i, 0)),
            pl.BlockSpec((blk, 16), lambda i: (i, 0)),
            pl.BlockSpec((blk, 16), lambda i: (i, 0)),
        ],
        out_shape=[
            jax.ShapeDtypeStruct((n, 16), jnp.float32),
            jax.ShapeDtypeStruct((n, 16), jnp.float32),
            jax.ShapeDtypeStruct((n, 16), jnp.float32),
        ],
    )(x, sum1, w1l, w1r, w2lp, w2rp, b1, b2p)


def _tc_combine(sum2, rc16, hr, n):
    def body(s2_ref, rc_ref, hr_ref, o_ref):
        o_ref[...] = (s2_ref[0] + s2_ref[1]) * rc_ref[...] + hr_ref[...]

    return pl.pallas_call(
        body,
        grid=(1,),
        in_specs=[
            pl.BlockSpec((2, n, 16), lambda i: (0, 0, 0)),
            pl.BlockSpec((n, 16), lambda i: (0, 0)),
            pl.BlockSpec((n, 16), lambda i: (0, 0)),
        ],
        out_specs=pl.BlockSpec((n, 16), lambda i: (0, 0)),
        out_shape=jax.ShapeDtypeStruct((n, 16), jnp.float32),
    )(sum2, rc16, hr)


def kernel(x, edge_index, W1l, b1, W1r, W2l, b2, W2r):
    n, d = x.shape
    e = edge_index.shape[1]
    o = W2l.shape[0]
    dp = d + 16

    src = edge_index[0]
    dst = edge_index[1]

    xx = jnp.concatenate(
        [x, jnp.ones((n, 1), jnp.float32), jnp.zeros((n, 15), jnp.float32)], axis=1)
    npad = -(-n // _EU) * _EU
    zero1 = jnp.zeros((npad // _NS, dp), jnp.float32)
    zero2 = jnp.zeros((npad // _NS, 16), jnp.float32)
    w2lp = jnp.zeros((16, d), jnp.float32).at[:o].set(W2l)
    w2rp = jnp.zeros((16, d), jnp.float32).at[:o].set(W2r)
    b2p = jnp.zeros((1, 16), jnp.float32).at[0, :o].set(b2)
    b1r = b1.reshape(1, d)

    seg1 = _make_seg_sum(n, e, dp, "sage_seg_sum_l1")
    seg2 = _make_seg_sum(n, e, 16, "sage_seg_sum_l2")

    sum1 = seg1(xx, src, dst, zero1)
    y2, hr, rc16 = _tc_dense(x, sum1, W1l, W1r, w2lp, w2rp, b1r, b2p, n, 1000)
    sum2 = seg2(y2, src, dst, zero2)
    outp = _tc_combine(sum2, rc16, hr, n)
    return outp[:, :o]
